# Initial kernel scaffold; baseline (speedup 1.0000x reference)
import jax
import jax.numpy as jnp
from jax.experimental import pallas as pl


def kernel(nodes, senders, receivers, n_node, is_root_mask, W0, b0, W1, b1, Wg, bg):
    raise NotImplementedError("write your pallas kernel here")


# speedup vs baseline: 12.7946x; 12.7946x over previous
import functools

import jax
import jax.numpy as jnp
from jax import lax
from jax.experimental import pallas as pl
from jax.experimental.pallas import tpu as pltpu
from jax.experimental.pallas import tpu_sc as plsc

NC = 2
NS = 16
K = 80


def _sc_aggregate(x, s3d, r3d):
    n, d = x.shape
    ch = s3d.shape[1]
    k = s3d.shape[2]
    rpt = ((n // NS) + 7) // 8 * 8
    rlast = n - (NS - 1) * rpt

    mesh = plsc.VectorSubcoreMesh(core_axis_name="c", subcore_axis_name="s")

    @functools.partial(
        pl.kernel,
        mesh=mesh,
        out_type=jax.ShapeDtypeStruct((NC, n, d), jnp.float32),
        scratch_types=[
            pltpu.VMEM((ch, k), jnp.int32),
            pltpu.VMEM((ch, k), jnp.int32),
            pltpu.VMEM((k, d), jnp.float32),
            pltpu.VMEM_SHARED((n, d), jnp.float32),
        ],
    )
    def run(x_hbm, s_hbm, r_hbm, out_hbm, sidx, ridx, rows, acc):
        c = lax.axis_index("c")
        s = lax.axis_index("s")
        rbase = s * rpt

        @pl.when(s < NS - 1)
        def _():
            pltpu.sync_copy(x_hbm.at[pl.ds(rbase, rpt)],
                            acc.at[pl.ds(rbase, rpt)])

        @pl.when(s == NS - 1)
        def _():
            pltpu.sync_copy(x_hbm.at[pl.ds(rbase, rlast)],
                            acc.at[pl.ds(rbase, rlast)])

        wid = c * NS + s
        pltpu.sync_copy(s_hbm.at[wid], sidx)
        pltpu.sync_copy(r_hbm.at[wid], ridx)
        plsc.subcore_barrier()

        def body(i, carry):
            pltpu.sync_copy(x_hbm.at[sidx.at[i]], rows)
            pltpu.sync_copy(rows, acc.at[ridx.at[i]], add=True)
            return carry

        lax.fori_loop(0, ch, body, 0)
        plsc.subcore_barrier()

        @pl.when(s < NS - 1)
        def _():
            pltpu.sync_copy(acc.at[pl.ds(rbase, rpt)],
                            out_hbm.at[c, pl.ds(rbase, rpt)])

        @pl.when(s == NS - 1)
        def _():
            pltpu.sync_copy(acc.at[pl.ds(rbase, rlast)],
                            out_hbm.at[c, pl.ds(rbase, rlast)])

    return run(x, s3d, r3d)


def _dense0(y_ref, nodes_ref, w0_ref, b0_ref, w1a_ref, w1b_ref, b1_ref,
            p0_ref, r0_ref):
    agg0 = y_ref[0] + y_ref[1] - nodes_ref[...]
    h0 = jnp.maximum(agg0 @ w0_ref[...] + b0_ref[...], 0.0)
    p0 = h0 @ w1a_ref[...]
    p0_ref[...] = p0
    r0_ref[...] = agg0 @ w1b_ref[...] + b1_ref[...] - p0


def _dense1(z_ref, r0_ref, mask_ref, starts_ref, ends_ref, wg_ref, bg_ref,
            out_ref):
    g = out_ref.shape[0]
    n = r0_ref.shape[0]
    h1 = jnp.maximum(z_ref[0] + z_ref[1] + r0_ref[...], 0.0)
    col = lax.broadcasted_iota(jnp.int32, (g, n), 1)
    sel = (col >= starts_ref[...]) & (col < ends_ref[...])
    onehot = jnp.where(sel, mask_ref[...], 0.0)
    hg = jnp.dot(onehot, h1, preferred_element_type=jnp.float32)
    out_ref[...] = hg @ wg_ref[...] + bg_ref[...]


def kernel(nodes, senders, receivers, n_node, is_root_mask,
           W0, b0, W1, b1, Wg, bg):
    n, d = nodes.shape
    g = n_node.shape[0]
    out_d = Wg.shape[1]

    e = senders.shape[0]
    ch = e // (NC * NS * K)
    s3d = senders.reshape(NC * NS, ch, K)
    r3d = receivers.reshape(NC * NS, ch, K)
    w1a = W1[:d]
    w1b = W1[d:]
    maskf = is_root_mask.astype(jnp.float32).reshape(1, n)
    ends = jnp.cumsum(n_node).reshape(g, 1)
    starts = ends - n_node.reshape(g, 1)

    y = _sc_aggregate(nodes, s3d, r3d)

    p0, r0 = pl.pallas_call(
        _dense0,
        out_shape=(jax.ShapeDtypeStruct((n, d), jnp.float32),
                   jax.ShapeDtypeStruct((n, d), jnp.float32)),
    )(y, nodes, W0, b0.reshape(1, -1), w1a, w1b, b1.reshape(1, -1))

    z = _sc_aggregate(p0, s3d, r3d)

    out = pl.pallas_call(
        _dense1,
        out_shape=jax.ShapeDtypeStruct((g, out_d), jnp.float32),
    )(z, r0, maskf, starts, ends, Wg, bg.reshape(1, -1))
    return out


# speedup vs baseline: 22.6119x; 1.7673x over previous
import functools

import jax
import jax.numpy as jnp
from jax import lax
from jax.experimental import pallas as pl
from jax.experimental.pallas import tpu as pltpu
from jax.experimental.pallas import tpu_sc as plsc

NC = 2
NS = 16
K = 125


def _sc_aggregate(x, sr):
    n, d = x.shape
    ch = sr.shape[1]
    k = sr.shape[3]
    rpt = ((n // NS) + 7) // 8 * 8
    rlast = n - (NS - 1) * rpt

    mesh = plsc.VectorSubcoreMesh(core_axis_name="c", subcore_axis_name="s")

    @functools.partial(
        pl.kernel,
        mesh=mesh,
        out_type=jax.ShapeDtypeStruct((NC, n, d), jnp.float32),
        scratch_types=[
            [pltpu.VMEM((2, k), jnp.int32) for _ in range(4)],
            [pltpu.VMEM((k, d), jnp.float32) for _ in range(2)],
            pltpu.VMEM_SHARED((n, d), jnp.float32),
            [pltpu.SemaphoreType.DMA for _ in range(4)],
            [pltpu.SemaphoreType.DMA for _ in range(2)],
        ],
    )
    def run(x_hbm, sr_hbm, out_hbm, idx, rows, acc, si, sg):
        c = lax.axis_index("c")
        s = lax.axis_index("s")
        rbase = s * rpt

        @pl.when(s < NS - 1)
        def _():
            pltpu.sync_copy(x_hbm.at[pl.ds(rbase, rpt)],
                            acc.at[pl.ds(rbase, rpt)])

        @pl.when(s == NS - 1)
        def _():
            pltpu.sync_copy(x_hbm.at[pl.ds(rbase, rlast)],
                            acc.at[pl.ds(rbase, rlast)])

        wid = c * NS + s
        plsc.subcore_barrier()

        pltpu.async_copy(sr_hbm.at[wid, 0], idx[0], si[0])
        pltpu.async_copy(sr_hbm.at[wid, 1], idx[1], si[1])

        def chunk_step(i, t):
            ia, ip, inx = idx[t % 4], idx[(t - 1) % 4], idx[(t + 2) % 4]
            sia, sin = si[t % 4], si[(t + 2) % 4]
            ra, rp = rows[t % 2], rows[(t - 1) % 2]
            sga, sgp = sg[t % 2], sg[(t - 1) % 2]
            pltpu.make_async_copy(sr_hbm.at[wid, i], ia, sia).wait()
            pltpu.async_copy(x_hbm.at[ia.at[0]], ra, sga)

            @pl.when(i + 2 < ch)
            def _():
                pltpu.async_copy(sr_hbm.at[wid, i + 2], inx, sin)

            @pl.when(i > 0)
            def _():
                pltpu.make_async_copy(x_hbm.at[ip.at[0]], rp, sgp).wait()
                pltpu.sync_copy(rp, acc.at[ip.at[1]], add=True)

        def body(j, carry):
            for t in range(4):
                chunk_step(4 * j + t, t)
            return carry

        lax.fori_loop(0, ch // 4, body, 0)
        il, rl, sl = idx[(ch - 1) % 4], rows[(ch - 1) % 2], sg[(ch - 1) % 2]
        pltpu.make_async_copy(x_hbm.at[il.at[0]], rl, sl).wait()
        pltpu.sync_copy(rl, acc.at[il.at[1]], add=True)
        plsc.subcore_barrier()

        @pl.when(s < NS - 1)
        def _():
            pltpu.sync_copy(acc.at[pl.ds(rbase, rpt)],
                            out_hbm.at[c, pl.ds(rbase, rpt)])

        @pl.when(s == NS - 1)
        def _():
            pltpu.sync_copy(acc.at[pl.ds(rbase, rlast)],
                            out_hbm.at[c, pl.ds(rbase, rlast)])

    return run(x, sr)


def _dense0(y_ref, nodes_ref, w0_ref, b0_ref, w1a_ref, w1b_ref, b1_ref,
            p0_ref, r0_ref):
    agg0 = y_ref[0] + y_ref[1] - nodes_ref[...]
    h0 = jnp.maximum(agg0 @ w0_ref[...] + b0_ref[...], 0.0)
    p0 = h0 @ w1a_ref[...]
    p0_ref[...] = p0
    r0_ref[...] = agg0 @ w1b_ref[...] + b1_ref[...] - p0


def _dense1(z_ref, r0_ref, mask_ref, starts_ref, ends_ref, wg_ref, bg_ref,
            out_ref):
    g = out_ref.shape[0]
    n = r0_ref.shape[0]
    h1 = jnp.maximum(z_ref[0] + z_ref[1] + r0_ref[...], 0.0)
    col = lax.broadcasted_iota(jnp.int32, (g, n), 1)
    sel = (col >= starts_ref[...]) & (col < ends_ref[...])
    onehot = jnp.where(sel, mask_ref[...], 0.0)
    hg = jnp.dot(onehot, h1, preferred_element_type=jnp.float32)
    out_ref[...] = hg @ wg_ref[...] + bg_ref[...]


def kernel(nodes, senders, receivers, n_node, is_root_mask,
           W0, b0, W1, b1, Wg, bg):
    n, d = nodes.shape
    g = n_node.shape[0]
    out_d = Wg.shape[1]

    e = senders.shape[0]
    ch = e // (NC * NS * K)
    sr = jnp.stack([senders.reshape(NC * NS, ch, K),
                    receivers.reshape(NC * NS, ch, K)], axis=2)
    w1a = W1[:d]
    w1b = W1[d:]
    maskf = is_root_mask.astype(jnp.float32).reshape(1, n)
    ends = jnp.cumsum(n_node).reshape(g, 1)
    starts = ends - n_node.reshape(g, 1)

    y = _sc_aggregate(nodes, sr)

    p0, r0 = pl.pallas_call(
        _dense0,
        out_shape=(jax.ShapeDtypeStruct((n, d), jnp.float32),
                   jax.ShapeDtypeStruct((n, d), jnp.float32)),
    )(y, nodes, W0, b0.reshape(1, -1), w1a, w1b, b1.reshape(1, -1))

    z = _sc_aggregate(p0, sr)

    out = pl.pallas_call(
        _dense1,
        out_shape=jax.ShapeDtypeStruct((g, out_d), jnp.float32),
    )(z, r0, maskf, starts, ends, Wg, bg.reshape(1, -1))
    return out
